# baseline (speedup 1.0000x reference)
import functools

import jax
import jax.numpy as jnp
from jax import lax
from jax.experimental import pallas as pl
from jax.experimental.pallas import tpu as pltpu
from jax.experimental.pallas import tpu_sc as plsc

_C, _T, _H, _W = 3, 300, 224, 224
_N = 32
_ROW = _H * _W

_IDX = tuple(j * (_T - 1) // (_N - 1) for j in range(_N))

_NC, _NS = 2, 16
_NW = _NC * _NS
_TASKS = _C * _N
_PER_W = _TASKS // _NW

_mesh = plsc.VectorSubcoreMesh(core_axis_name="c", subcore_axis_name="s")


@functools.partial(
    pl.kernel,
    out_type=jax.ShapeDtypeStruct((_TASKS, _ROW), jnp.float32),
    mesh=_mesh,
    scratch_types=[pltpu.SemaphoreType.DMA],
)
def _sc_gather(x_hbm, out_hbm, sem):
    wid = lax.axis_index("s") * _NC + lax.axis_index("c")
    for w in range(_NW):

        @pl.when(wid == w)
        def _copies(w=w):
            descs = []
            for k in range(_PER_W):
                t = w * _PER_W + k
                c, j = divmod(t, _N)
                src = c * _T + _IDX[j]
                descs.append(
                    pltpu.make_async_copy(x_hbm.at[src], out_hbm.at[t], sem)
                )
            for d in descs:
                d.start()
            for d in descs:
                d.wait()


def kernel(x):
    x2 = x.reshape(_C * _T, _ROW)
    out2 = _sc_gather(x2)
    return out2.reshape(_C, _N, _H, _W)


# speedup vs baseline: 3.2536x; 3.2536x over previous
import functools

import jax
import jax.numpy as jnp
from jax import lax
from jax.experimental import pallas as pl
from jax.experimental.pallas import tpu as pltpu
from jax.experimental.pallas import tpu_sc as plsc

_C, _T, _H, _W = 3, 300, 224, 224
_N = 32
_ROW = _H * _W

_IDX = tuple(j * (_T - 1) // (_N - 1) for j in range(_N))

_NC, _NS = 2, 16
_NW = _NC * _NS
_TASKS = _C * _N
_PER_W = _TASKS // _NW

_mesh = plsc.VectorSubcoreMesh(core_axis_name="c", subcore_axis_name="s")


@functools.partial(
    pl.kernel,
    out_type=jax.ShapeDtypeStruct((_TASKS, _ROW), jnp.float32),
    mesh=_mesh,
    scratch_types=[
        pltpu.VMEM((2, _ROW), jnp.float32),
        pltpu.SemaphoreType.DMA((2,)),
        pltpu.SemaphoreType.DMA((2,)),
    ],
)
def _sc_gather(x_hbm, out_hbm, buf, isem, osem):
    wid = lax.axis_index("s") * _NC + lax.axis_index("c")
    for w in range(_NW):

        @pl.when(wid == w)
        def _copies(w=w):
            ins, outs = [], []
            for k in range(_PER_W):
                t = w * _PER_W + k
                c, j = divmod(t, _N)
                src = c * _T + _IDX[j]
                b = k % 2
                ins.append(
                    pltpu.make_async_copy(x_hbm.at[src], buf.at[b], isem.at[b])
                )
                outs.append(
                    pltpu.make_async_copy(buf.at[b], out_hbm.at[t], osem.at[b])
                )
            ins[0].start()
            ins[1].start()
            ins[0].wait()
            outs[0].start()
            ins[1].wait()
            outs[1].start()
            outs[0].wait()
            ins[2].start()
            ins[2].wait()
            outs[2].start()
            outs[1].wait()
            outs[2].wait()


def kernel(x):
    x2 = x.reshape(_C * _T, _ROW)
    out2 = _sc_gather(x2)
    return out2.reshape(_C, _N, _H, _W)


# speedup vs baseline: 23.3406x; 7.1737x over previous
import functools

import jax
import jax.numpy as jnp
from jax import lax
from jax.experimental import pallas as pl
from jax.experimental.pallas import tpu as pltpu
from jax.experimental.pallas import tpu_sc as plsc

_C, _T, _H, _W = 3, 300, 224, 224
_N = 32

_NC, _NS = 2, 16
_NW = _NC * _NS
_TASKS = _C * _N
_PER_W = _TASKS // _NW

_mesh = plsc.VectorSubcoreMesh(core_axis_name="c", subcore_axis_name="s")


@functools.partial(
    pl.kernel,
    out_type=jax.ShapeDtypeStruct((_C, _N, _H, _W), jnp.float32),
    mesh=_mesh,
    scratch_types=[
        pltpu.VMEM((2, _H, _W), jnp.float32),
        pltpu.SemaphoreType.DMA((2,)),
        pltpu.SemaphoreType.DMA((2,)),
    ],
    compiler_params=pltpu.CompilerParams(use_tc_tiling_on_sc=True),
)
def _sc_gather(x_hbm, out_hbm, buf, isem, osem):
    wid = lax.axis_index("s") * _NC + lax.axis_index("c")
    ins, outs = [], []
    for k in range(_PER_W):
        t = wid * _PER_W + k
        c = t >> 5
        j = t & (_N - 1)
        src = (j * (299 * 33826)) >> 20
        b = k % 2
        ins.append(pltpu.make_async_copy(x_hbm.at[c, src], buf.at[b], isem.at[b]))
        outs.append(pltpu.make_async_copy(buf.at[b], out_hbm.at[c, j], osem.at[b]))
    ins[0].start()
    ins[1].start()
    ins[0].wait()
    outs[0].start()
    ins[1].wait()
    outs[1].start()
    outs[0].wait()
    ins[2].start()
    ins[2].wait()
    outs[2].start()
    outs[1].wait()
    outs[2].wait()


def kernel(x):
    return _sc_gather(x)
